# baseline (speedup 1.0000x reference)
import functools

import jax
import jax.numpy as jnp
from jax import lax
from jax.experimental import pallas as pl
from jax.experimental.pallas import tpu as pltpu
from jax.experimental.pallas import tpu_sc as plsc

SRC_SHAPE = (16384, 1024)
N = SRC_SHAPE[0] * SRC_SHAPE[1]
NUM_WORKERS = 32
PER_WORKER = N // NUM_WORKERS
CHUNK = 16384
NUM_CHUNKS = PER_WORKER // CHUNK

_mesh = plsc.VectorSubcoreMesh(core_axis_name="c", subcore_axis_name="s")


@functools.partial(
    pl.kernel,
    mesh=_mesh,
    compiler_params=pltpu.CompilerParams(needs_layout_passes=False),
    out_type=jax.ShapeDtypeStruct((N // 4,), jnp.int32),
    scratch_types=[
        pltpu.VMEM((256,), jnp.float32),
        pltpu.VMEM((CHUNK,), jnp.float32),
        pltpu.VMEM((CHUNK,), jnp.float32),
        pltpu.VMEM((CHUNK,), jnp.int32),
        pltpu.VMEM((CHUNK,), jnp.int32),
        pltpu.VMEM((CHUNK // 4,), jnp.int32),
        pltpu.VMEM((CHUNK // 4,), jnp.int32),
        pltpu.SemaphoreType.DMA,
        pltpu.SemaphoreType.DMA,
        pltpu.SemaphoreType.DMA,
        pltpu.SemaphoreType.DMA,
        pltpu.SemaphoreType.DMA,
        pltpu.SemaphoreType.DMA,
    ],
)
def _bsgen_sc(src_hbm, table_hbm, idx_hbm, out_hbm,
              table_v, src_v0, src_v1, idx_v0, idx_v1, out_v0, out_v1,
              sem_s0, sem_s1, sem_i0, sem_i1, sem_o0, sem_o1):
    wid = lax.axis_index("s") * 2 + lax.axis_index("c")
    base = wid * PER_WORKER

    slots = (
        (src_v0, idx_v0, out_v0, sem_s0, sem_i0, sem_o0),
        (src_v1, idx_v1, out_v1, sem_s1, sem_i1, sem_o1),
    )

    def start_in(g, slot):
        src_v, idx_v, _, sem_s, sem_i, _ = slot
        off = base + g * CHUNK
        pltpu.async_copy(src_hbm.at[pl.ds(off, CHUNK)], src_v, sem_s)
        pltpu.async_copy(idx_hbm.at[pl.ds(off, CHUNK)], idx_v, sem_i)

    def wait_in(slot):
        src_v, idx_v, _, sem_s, sem_i, _ = slot
        pltpu.make_async_copy(src_hbm.at[pl.ds(base, CHUNK)], src_v, sem_s).wait()
        pltpu.make_async_copy(idx_hbm.at[pl.ds(base, CHUNK)], idx_v, sem_i).wait()

    base_w = wid * (PER_WORKER // 4)
    CHUNK_W = CHUNK // 4

    def start_out(g, slot):
        out_v, sem_o = slot[2], slot[5]
        pltpu.async_copy(out_v, out_hbm.at[pl.ds(base_w + g * CHUNK_W, CHUNK_W)],
                         sem_o)

    def wait_out(slot):
        out_v, sem_o = slot[2], slot[5]
        pltpu.make_async_copy(out_v, out_hbm.at[pl.ds(base_w, CHUNK_W)],
                              sem_o).wait()

    pltpu.sync_copy(table_hbm, table_v)

    iota4 = lax.iota(jnp.int32, 16) * 4

    def compute(slot):
        src_v, idx_v, out_v = slot[0], slot[1], slot[2]

        def inner(j, carry):
            off = j * 64
            sbase = iota4 + off
            acc = None
            for m in range(4):
                pos = [sbase + m] if m else [sbase]
                im = plsc.load_gather(idx_v, pos)
                tm = plsc.load_gather(table_v, [im])
                sm = plsc.load_gather(src_v, pos)
                rm = jnp.where(sm > tm, jnp.int32(1 << (8 * m)), jnp.int32(0))
                acc = rm if acc is None else acc | rm
            out_v[pl.ds(j * 16, 16)] = acc
            return carry

        lax.fori_loop(0, CHUNK // 64, inner, 0)

    for b in range(2):
        start_in(b, slots[b])

    def pair_body(p, carry):
        for b in range(2):
            g = p * 2 + b
            slot = slots[b]
            wait_in(slot)

            @pl.when(g >= 2)
            def _():
                wait_out(slot)

            compute(slot)
            start_out(g, slot)

            @pl.when(g + 2 < NUM_CHUNKS)
            def _():
                start_in(g + 2, slot)
        return carry

    lax.fori_loop(0, NUM_CHUNKS // 2, pair_body, 0)

    for b in range(2):
        wait_out(slots[b])


def kernel(source, rng_seq, rng_idx):
    src = source.reshape(N)
    idx = rng_idx.reshape(N).astype(jnp.int32)
    out_w = _bsgen_sc(src, rng_seq, idx)
    out8 = lax.bitcast_convert_type(out_w, jnp.uint8).astype(jnp.int8)
    return out8.reshape(SRC_SHAPE)


# speedup vs baseline: 1.3771x; 1.3771x over previous
import functools

import jax
import jax.numpy as jnp
from jax import lax
from jax.experimental import pallas as pl
from jax.experimental.pallas import tpu as pltpu
from jax.experimental.pallas import tpu_sc as plsc

SRC_SHAPE = (16384, 1024)
ROWS, COLS = SRC_SHAPE
COLS_W = COLS // 4
NUM_WORKERS = 32
ROWS_PER_WORKER = ROWS // NUM_WORKERS
CHUNK_ROWS = 16
NUM_CHUNKS = ROWS_PER_WORKER // CHUNK_ROWS
CHUNK = CHUNK_ROWS * COLS

_mesh = plsc.VectorSubcoreMesh(core_axis_name="c", subcore_axis_name="s")


@functools.partial(
    pl.kernel,
    mesh=_mesh,
    compiler_params=pltpu.CompilerParams(needs_layout_passes=False),
    out_type=jax.ShapeDtypeStruct((ROWS, COLS_W), jnp.int32),
    scratch_types=[
        pltpu.VMEM((256,), jnp.float32),
        pltpu.VMEM((CHUNK_ROWS, COLS), jnp.float32),
        pltpu.VMEM((CHUNK_ROWS, COLS), jnp.float32),
        pltpu.VMEM((CHUNK_ROWS, COLS), jnp.int32),
        pltpu.VMEM((CHUNK_ROWS, COLS), jnp.int32),
        pltpu.VMEM((CHUNK_ROWS, COLS_W), jnp.int32),
        pltpu.VMEM((CHUNK_ROWS, COLS_W), jnp.int32),
        pltpu.SemaphoreType.DMA,
        pltpu.SemaphoreType.DMA,
        pltpu.SemaphoreType.DMA,
        pltpu.SemaphoreType.DMA,
        pltpu.SemaphoreType.DMA,
        pltpu.SemaphoreType.DMA,
    ],
)
def _bsgen_sc(src_hbm, table_hbm, idx_hbm, out_hbm,
              table_v, src_v0, src_v1, idx_v0, idx_v1, out_v0, out_v1,
              sem_s0, sem_s1, sem_i0, sem_i1, sem_o0, sem_o1):
    wid = lax.axis_index("s") * 2 + lax.axis_index("c")
    row0 = wid * ROWS_PER_WORKER

    slots = (
        (src_v0, idx_v0, out_v0, sem_s0, sem_i0, sem_o0),
        (src_v1, idx_v1, out_v1, sem_s1, sem_i1, sem_o1),
    )

    def start_in(g, slot):
        src_v, idx_v, _, sem_s, sem_i, _ = slot
        r = row0 + g * CHUNK_ROWS
        pltpu.async_copy(src_hbm.at[pl.ds(r, CHUNK_ROWS), :], src_v, sem_s)
        pltpu.async_copy(idx_hbm.at[pl.ds(r, CHUNK_ROWS), :], idx_v, sem_i)

    def wait_in(slot):
        src_v, idx_v, _, sem_s, sem_i, _ = slot
        pltpu.make_async_copy(src_hbm.at[pl.ds(row0, CHUNK_ROWS), :], src_v,
                              sem_s).wait()
        pltpu.make_async_copy(idx_hbm.at[pl.ds(row0, CHUNK_ROWS), :], idx_v,
                              sem_i).wait()

    def start_out(g, slot):
        out_v, sem_o = slot[2], slot[5]
        r = row0 + g * CHUNK_ROWS
        pltpu.async_copy(out_v, out_hbm.at[pl.ds(r, CHUNK_ROWS), :], sem_o)

    def wait_out(slot):
        out_v, sem_o = slot[2], slot[5]
        pltpu.make_async_copy(out_v, out_hbm.at[pl.ds(row0, CHUNK_ROWS), :],
                              sem_o).wait()

    pltpu.sync_copy(table_hbm, table_v)

    iota4 = lax.iota(jnp.int32, 16) * 4

    def compute(slot):
        src_v, idx_v, out_v = slot[0], slot[1], slot[2]

        def inner(j, carry):
            r = j // 16
            cbase = (j % 16) * 64
            rv = jnp.full((16,), r, jnp.int32)
            col0 = iota4 + cbase
            acc = None
            for m in range(4):
                cols = [rv, col0 + m] if m else [rv, col0]
                im = plsc.load_gather(idx_v, cols)
                tm = plsc.load_gather(table_v, [im])
                sm = plsc.load_gather(src_v, cols)
                rm = jnp.where(sm > tm, jnp.int32(1 << (8 * m)), jnp.int32(0))
                acc = rm if acc is None else acc | rm
            out_v[r, pl.ds((j % 16) * 16, 16)] = acc
            return carry

        lax.fori_loop(0, CHUNK // 64, inner, 0)

    for b in range(2):
        start_in(b, slots[b])

    def pair_body(p, carry):
        for b in range(2):
            g = p * 2 + b
            slot = slots[b]
            wait_in(slot)

            @pl.when(g >= 2)
            def _():
                wait_out(slot)

            compute(slot)
            start_out(g, slot)

            @pl.when(g + 2 < NUM_CHUNKS)
            def _():
                start_in(g + 2, slot)
        return carry

    lax.fori_loop(0, NUM_CHUNKS // 2, pair_body, 0)

    for b in range(2):
        wait_out(slots[b])


def kernel(source, rng_seq, rng_idx):
    idx = rng_idx.astype(jnp.int32)
    out_w = _bsgen_sc(source, rng_seq, idx)
    out8 = lax.bitcast_convert_type(out_w, jnp.uint8).astype(jnp.int8)
    return out8.reshape(SRC_SHAPE)


# speedup vs baseline: 2.9590x; 2.1488x over previous
import functools

import jax
import jax.numpy as jnp
from jax import lax
from jax.experimental import pallas as pl
from jax.experimental.pallas import tpu as pltpu
from jax.experimental.pallas import tpu_sc as plsc

SRC_SHAPE = (16384, 1024)
ROWS, COLS = SRC_SHAPE
WROWS = ROWS // 4
NUM_WORKERS = 32
ROWS_PER_WORKER = ROWS // NUM_WORKERS
CHUNK_ROWS = 32
CHUNK_COLS = 512
CHUNK_WROWS = CHUNK_ROWS // 4
NUM_CHUNKS = (ROWS_PER_WORKER // CHUNK_ROWS) * (COLS // CHUNK_COLS)
CHUNK = CHUNK_ROWS * CHUNK_COLS

_mesh = plsc.VectorSubcoreMesh(core_axis_name="c", subcore_axis_name="s")


@functools.partial(
    pl.kernel,
    mesh=_mesh,
    compiler_params=pltpu.CompilerParams(needs_layout_passes=False),
    out_type=jax.ShapeDtypeStruct((WROWS, COLS), jnp.int32),
    scratch_types=[
        pltpu.VMEM((256,), jnp.float32),
        pltpu.VMEM((CHUNK_ROWS, CHUNK_COLS), jnp.float32),
        pltpu.VMEM((CHUNK_ROWS, CHUNK_COLS), jnp.float32),
        pltpu.VMEM((CHUNK_ROWS, CHUNK_COLS), jnp.int32),
        pltpu.VMEM((CHUNK_ROWS, CHUNK_COLS), jnp.int32),
        pltpu.VMEM((CHUNK_WROWS, CHUNK_COLS), jnp.int32),
        pltpu.VMEM((CHUNK_WROWS, CHUNK_COLS), jnp.int32),
        pltpu.SemaphoreType.DMA,
        pltpu.SemaphoreType.DMA,
        pltpu.SemaphoreType.DMA,
        pltpu.SemaphoreType.DMA,
        pltpu.SemaphoreType.DMA,
        pltpu.SemaphoreType.DMA,
    ],
)
def _bsgen_sc(src_hbm, table_hbm, idx_hbm, out_hbm,
              table_v, src_v0, src_v1, idx_v0, idx_v1, out_v0, out_v1,
              sem_s0, sem_s1, sem_i0, sem_i1, sem_o0, sem_o1):
    wid = lax.axis_index("s") * 2 + lax.axis_index("c")
    row0 = wid * ROWS_PER_WORKER
    wrow0 = wid * (ROWS_PER_WORKER // 4)

    slots = (
        (src_v0, idx_v0, out_v0, sem_s0, sem_i0, sem_o0),
        (src_v1, idx_v1, out_v1, sem_s1, sem_i1, sem_o1),
    )

    def start_in(g, slot):
        src_v, idx_v, _, sem_s, sem_i, _ = slot
        r = row0 + (g // 2) * CHUNK_ROWS
        cb = (g % 2) * CHUNK_COLS
        pltpu.async_copy(
            src_hbm.at[pl.ds(r, CHUNK_ROWS), pl.ds(cb, CHUNK_COLS)],
            src_v, sem_s)
        pltpu.async_copy(
            idx_hbm.at[pl.ds(r, CHUNK_ROWS), pl.ds(cb, CHUNK_COLS)],
            idx_v, sem_i)

    def wait_in(slot):
        src_v, idx_v, _, sem_s, sem_i, _ = slot
        pltpu.make_async_copy(
            src_hbm.at[pl.ds(row0, CHUNK_ROWS), pl.ds(0, CHUNK_COLS)],
            src_v, sem_s).wait()
        pltpu.make_async_copy(
            idx_hbm.at[pl.ds(row0, CHUNK_ROWS), pl.ds(0, CHUNK_COLS)],
            idx_v, sem_i).wait()

    def start_out(g, slot):
        out_v, sem_o = slot[2], slot[5]
        r = wrow0 + (g // 2) * CHUNK_WROWS
        cb = (g % 2) * CHUNK_COLS
        pltpu.async_copy(
            out_v, out_hbm.at[pl.ds(r, CHUNK_WROWS), pl.ds(cb, CHUNK_COLS)],
            sem_o)

    def wait_out(slot):
        out_v, sem_o = slot[2], slot[5]
        pltpu.make_async_copy(
            out_v, out_hbm.at[pl.ds(wrow0, CHUNK_WROWS), pl.ds(0, CHUNK_COLS)],
            sem_o).wait()

    pltpu.sync_copy(table_hbm, table_v)

    def compute(slot):
        src_v, idx_v, out_v = slot[0], slot[1], slot[2]

        NCG = CHUNK_COLS // 16

        def inner(j, carry):
            wr = j // NCG
            c0 = (j % NCG) * 16
            acc = None
            for b in range(4):
                r = 4 * wr + b
                iv = idx_v[r, pl.ds(c0, 16)]
                tv = plsc.load_gather(table_v, [iv])
                sv = src_v[r, pl.ds(c0, 16)]
                rm = jnp.where(sv > tv, jnp.int32(1 << (8 * b)), jnp.int32(0))
                acc = rm if acc is None else acc | rm
            out_v[wr, pl.ds(c0, 16)] = acc
            return carry

        lax.fori_loop(0, CHUNK // 64, inner, 0)

    for b in range(2):
        start_in(b, slots[b])

    def pair_body(p, carry):
        for b in range(2):
            g = p * 2 + b
            slot = slots[b]
            wait_in(slot)

            @pl.when(g >= 2)
            def _():
                wait_out(slot)

            compute(slot)
            start_out(g, slot)

            @pl.when(g + 2 < NUM_CHUNKS)
            def _():
                start_in(g + 2, slot)
        return carry

    lax.fori_loop(0, NUM_CHUNKS // 2, pair_body, 0)

    for b in range(2):
        wait_out(slots[b])


_TC_BLOCK_WROWS = 512


def _unpack_body(w_ref, o_ref):
    o_ref[...] = pltpu.bitcast(w_ref[...], jnp.int8)


_unpack_tc = pl.pallas_call(
    _unpack_body,
    grid=(WROWS // _TC_BLOCK_WROWS,),
    in_specs=[pl.BlockSpec((_TC_BLOCK_WROWS, COLS), lambda i: (i, 0))],
    out_specs=pl.BlockSpec((4 * _TC_BLOCK_WROWS, COLS), lambda i: (i, 0)),
    out_shape=jax.ShapeDtypeStruct(SRC_SHAPE, jnp.int8),
)


def kernel(source, rng_seq, rng_idx):
    idx = rng_idx.astype(jnp.int32)
    out_w = _bsgen_sc(source, rng_seq, idx)
    return _unpack_tc(out_w)


# speedup vs baseline: 4.1323x; 1.3965x over previous
import functools

import jax
import jax.numpy as jnp
from jax import lax
from jax.experimental import pallas as pl
from jax.experimental.pallas import tpu as pltpu
from jax.experimental.pallas import tpu_sc as plsc

SRC_SHAPE = (16384, 1024)
ROWS, COLS = SRC_SHAPE
WROWS = ROWS // 4
NUM_WORKERS = 32
ROWS_PER_WORKER = ROWS // NUM_WORKERS
CHUNK_ROWS = 32
CHUNK_COLS = 512
CHUNK_WROWS = CHUNK_ROWS // 4
NUM_CHUNKS = (ROWS_PER_WORKER // CHUNK_ROWS) * (COLS // CHUNK_COLS)
CHUNK = CHUNK_ROWS * CHUNK_COLS

_mesh = plsc.VectorSubcoreMesh(core_axis_name="c", subcore_axis_name="s")


@functools.partial(
    pl.kernel,
    mesh=_mesh,
    compiler_params=pltpu.CompilerParams(needs_layout_passes=False),
    out_type=jax.ShapeDtypeStruct((WROWS, COLS), jnp.int32),
    scratch_types=[
        pltpu.VMEM((256,), jnp.float32),
        pltpu.VMEM((CHUNK_ROWS, CHUNK_COLS), jnp.float32),
        pltpu.VMEM((CHUNK_ROWS, CHUNK_COLS), jnp.float32),
        pltpu.VMEM((CHUNK_ROWS, CHUNK_COLS), jnp.int32),
        pltpu.VMEM((CHUNK_ROWS, CHUNK_COLS), jnp.int32),
        pltpu.VMEM((CHUNK_WROWS, CHUNK_COLS), jnp.int32),
        pltpu.VMEM((CHUNK_WROWS, CHUNK_COLS), jnp.int32),
        pltpu.SemaphoreType.DMA,
        pltpu.SemaphoreType.DMA,
        pltpu.SemaphoreType.DMA,
        pltpu.SemaphoreType.DMA,
        pltpu.SemaphoreType.DMA,
        pltpu.SemaphoreType.DMA,
    ],
)
def _bsgen_sc(src_hbm, table_hbm, idx_hbm, out_hbm,
              table_v, src_v0, src_v1, idx_v0, idx_v1, out_v0, out_v1,
              sem_s0, sem_s1, sem_i0, sem_i1, sem_o0, sem_o1):
    wid = lax.axis_index("s") * 2 + lax.axis_index("c")
    row0 = wid * ROWS_PER_WORKER
    wrow0 = wid * (ROWS_PER_WORKER // 4)

    slots = (
        (src_v0, idx_v0, out_v0, sem_s0, sem_i0, sem_o0),
        (src_v1, idx_v1, out_v1, sem_s1, sem_i1, sem_o1),
    )

    def start_in(g, slot):
        src_v, idx_v, _, sem_s, sem_i, _ = slot
        r = row0 + (g // 2) * CHUNK_ROWS
        cb = (g % 2) * CHUNK_COLS
        pltpu.async_copy(
            src_hbm.at[pl.ds(r, CHUNK_ROWS), pl.ds(cb, CHUNK_COLS)],
            src_v, sem_s)
        pltpu.async_copy(
            idx_hbm.at[pl.ds(r, CHUNK_ROWS), pl.ds(cb, CHUNK_COLS)],
            idx_v, sem_i)

    def wait_in(slot):
        src_v, idx_v, _, sem_s, sem_i, _ = slot
        pltpu.make_async_copy(
            src_hbm.at[pl.ds(row0, CHUNK_ROWS), pl.ds(0, CHUNK_COLS)],
            src_v, sem_s).wait()
        pltpu.make_async_copy(
            idx_hbm.at[pl.ds(row0, CHUNK_ROWS), pl.ds(0, CHUNK_COLS)],
            idx_v, sem_i).wait()

    def start_out(g, slot):
        out_v, sem_o = slot[2], slot[5]
        r = wrow0 + (g // 2) * CHUNK_WROWS
        cb = (g % 2) * CHUNK_COLS
        pltpu.async_copy(
            out_v, out_hbm.at[pl.ds(r, CHUNK_WROWS), pl.ds(cb, CHUNK_COLS)],
            sem_o)

    def wait_out(slot):
        out_v, sem_o = slot[2], slot[5]
        pltpu.make_async_copy(
            out_v, out_hbm.at[pl.ds(wrow0, CHUNK_WROWS), pl.ds(0, CHUNK_COLS)],
            sem_o).wait()

    pltpu.sync_copy(table_hbm, table_v)

    def compute(slot):
        src_v, idx_v, out_v = slot[0], slot[1], slot[2]

        NCG = CHUNK_COLS // 16

        @plsc.parallel_loop(0, CHUNK // 64, 1, unroll=4)
        def inner(j):
            wr = j // NCG
            c0 = (j % NCG) * 16
            acc = None
            for b in range(4):
                r = 4 * wr + b
                iv = idx_v[r, pl.ds(c0, 16)]
                tv = plsc.load_gather(table_v, [iv])
                sv = src_v[r, pl.ds(c0, 16)]
                rm = jnp.where(sv > tv, jnp.int32(1 << (8 * b)), jnp.int32(0))
                acc = rm if acc is None else acc | rm
            out_v[wr, pl.ds(c0, 16)] = acc

    for b in range(2):
        start_in(b, slots[b])

    def pair_body(p, carry):
        for b in range(2):
            g = p * 2 + b
            slot = slots[b]
            wait_in(slot)

            @pl.when(g >= 2)
            def _():
                wait_out(slot)

            compute(slot)
            start_out(g, slot)

            @pl.when(g + 2 < NUM_CHUNKS)
            def _():
                start_in(g + 2, slot)
        return carry

    lax.fori_loop(0, NUM_CHUNKS // 2, pair_body, 0)

    for b in range(2):
        wait_out(slots[b])


_TC_BLOCK_WROWS = 512


def _unpack_body(w_ref, o_ref):
    o_ref[...] = pltpu.bitcast(w_ref[...], jnp.int8)


_unpack_tc = pl.pallas_call(
    _unpack_body,
    grid=(WROWS // _TC_BLOCK_WROWS,),
    in_specs=[pl.BlockSpec((_TC_BLOCK_WROWS, COLS), lambda i: (i, 0))],
    out_specs=pl.BlockSpec((4 * _TC_BLOCK_WROWS, COLS), lambda i: (i, 0)),
    out_shape=jax.ShapeDtypeStruct(SRC_SHAPE, jnp.int8),
)


def kernel(source, rng_seq, rng_idx):
    idx = rng_idx.astype(jnp.int32)
    out_w = _bsgen_sc(source, rng_seq, idx)
    return _unpack_tc(out_w)


# speedup vs baseline: 4.1907x; 1.0142x over previous
import functools

import jax
import jax.numpy as jnp
from jax import lax
from jax.experimental import pallas as pl
from jax.experimental.pallas import tpu as pltpu
from jax.experimental.pallas import tpu_sc as plsc

SRC_SHAPE = (16384, 1024)
ROWS, COLS = SRC_SHAPE
WROWS = ROWS // 4
NUM_WORKERS = 32
ROWS_PER_WORKER = ROWS // NUM_WORKERS
CHUNK_ROWS = 32
CHUNK_COLS = 512
CHUNK_WROWS = CHUNK_ROWS // 4
NUM_CHUNKS = (ROWS_PER_WORKER // CHUNK_ROWS) * (COLS // CHUNK_COLS)
CHUNK = CHUNK_ROWS * CHUNK_COLS

_mesh = plsc.VectorSubcoreMesh(core_axis_name="c", subcore_axis_name="s")


@functools.partial(
    pl.kernel,
    mesh=_mesh,
    compiler_params=pltpu.CompilerParams(needs_layout_passes=False),
    out_type=jax.ShapeDtypeStruct((WROWS, COLS), jnp.int32),
    scratch_types=[
        pltpu.VMEM((256,), jnp.float32),
        pltpu.VMEM((CHUNK_ROWS, CHUNK_COLS), jnp.float32),
        pltpu.VMEM((CHUNK_ROWS, CHUNK_COLS), jnp.float32),
        pltpu.VMEM((CHUNK_ROWS, CHUNK_COLS), jnp.int32),
        pltpu.VMEM((CHUNK_ROWS, CHUNK_COLS), jnp.int32),
        pltpu.VMEM((CHUNK_WROWS, CHUNK_COLS), jnp.int32),
        pltpu.VMEM((CHUNK_WROWS, CHUNK_COLS), jnp.int32),
        pltpu.SemaphoreType.DMA,
        pltpu.SemaphoreType.DMA,
        pltpu.SemaphoreType.DMA,
        pltpu.SemaphoreType.DMA,
        pltpu.SemaphoreType.DMA,
        pltpu.SemaphoreType.DMA,
    ],
)
def _bsgen_sc(src_hbm, table_hbm, idx_hbm, out_hbm,
              table_v, src_v0, src_v1, idx_v0, idx_v1, out_v0, out_v1,
              sem_s0, sem_s1, sem_i0, sem_i1, sem_o0, sem_o1):
    wid = lax.axis_index("s") * 2 + lax.axis_index("c")
    row0 = wid * ROWS_PER_WORKER
    wrow0 = wid * (ROWS_PER_WORKER // 4)

    slots = (
        (src_v0, idx_v0, out_v0, sem_s0, sem_i0, sem_o0),
        (src_v1, idx_v1, out_v1, sem_s1, sem_i1, sem_o1),
    )

    def start_in(g, slot):
        src_v, idx_v, _, sem_s, sem_i, _ = slot
        r = row0 + (g // 2) * CHUNK_ROWS
        cb = (g % 2) * CHUNK_COLS
        pltpu.async_copy(
            src_hbm.at[pl.ds(r, CHUNK_ROWS), pl.ds(cb, CHUNK_COLS)],
            src_v, sem_s)
        pltpu.async_copy(
            idx_hbm.at[pl.ds(r, CHUNK_ROWS), pl.ds(cb, CHUNK_COLS)],
            idx_v, sem_i)

    def wait_in(slot):
        src_v, idx_v, _, sem_s, sem_i, _ = slot
        pltpu.make_async_copy(
            src_hbm.at[pl.ds(row0, CHUNK_ROWS), pl.ds(0, CHUNK_COLS)],
            src_v, sem_s).wait()
        pltpu.make_async_copy(
            idx_hbm.at[pl.ds(row0, CHUNK_ROWS), pl.ds(0, CHUNK_COLS)],
            idx_v, sem_i).wait()

    def start_out(g, slot):
        out_v, sem_o = slot[2], slot[5]
        r = wrow0 + (g // 2) * CHUNK_WROWS
        cb = (g % 2) * CHUNK_COLS
        pltpu.async_copy(
            out_v, out_hbm.at[pl.ds(r, CHUNK_WROWS), pl.ds(cb, CHUNK_COLS)],
            sem_o)

    def wait_out(slot):
        out_v, sem_o = slot[2], slot[5]
        pltpu.make_async_copy(
            out_v, out_hbm.at[pl.ds(wrow0, CHUNK_WROWS), pl.ds(0, CHUNK_COLS)],
            sem_o).wait()

    pltpu.sync_copy(table_hbm, table_v)

    def compute(slot):
        src_v, idx_v, out_v = slot[0], slot[1], slot[2]

        NCG = CHUNK_COLS // 16

        @plsc.parallel_loop(0, CHUNK // 64, 1, unroll=8)
        def inner(j):
            wr = j // NCG
            c0 = (j % NCG) * 16
            acc = None
            for b in range(4):
                r = 4 * wr + b
                iv = idx_v[r, pl.ds(c0, 16)]
                tv = plsc.load_gather(table_v, [iv])
                sv = src_v[r, pl.ds(c0, 16)]
                rm = jnp.where(sv > tv, jnp.int32(1 << (8 * b)), jnp.int32(0))
                acc = rm if acc is None else acc | rm
            out_v[wr, pl.ds(c0, 16)] = acc

    for b in range(2):
        start_in(b, slots[b])

    def pair_body(p, carry):
        for b in range(2):
            g = p * 2 + b
            slot = slots[b]
            wait_in(slot)

            @pl.when(g >= 2)
            def _():
                wait_out(slot)

            compute(slot)
            start_out(g, slot)

            @pl.when(g + 2 < NUM_CHUNKS)
            def _():
                start_in(g + 2, slot)
        return carry

    lax.fori_loop(0, NUM_CHUNKS // 2, pair_body, 0)

    for b in range(2):
        wait_out(slots[b])


_TC_BLOCK_WROWS = 512


def _unpack_body(w_ref, o_ref):
    o_ref[...] = pltpu.bitcast(w_ref[...], jnp.int8)


_unpack_tc = pl.pallas_call(
    _unpack_body,
    grid=(WROWS // _TC_BLOCK_WROWS,),
    in_specs=[pl.BlockSpec((_TC_BLOCK_WROWS, COLS), lambda i: (i, 0))],
    out_specs=pl.BlockSpec((4 * _TC_BLOCK_WROWS, COLS), lambda i: (i, 0)),
    out_shape=jax.ShapeDtypeStruct(SRC_SHAPE, jnp.int8),
)


def kernel(source, rng_seq, rng_idx):
    idx = rng_idx.astype(jnp.int32)
    out_w = _bsgen_sc(source, rng_seq, idx)
    return _unpack_tc(out_w)
